# baseline (speedup 1.0000x reference)
import jax
import jax.numpy as jnp
from jax import lax
from jax.experimental import pallas as pl
from jax.experimental.pallas import tpu as pltpu
from jax.experimental.pallas import tpu_sc as plsc

B, T, N, F_IN, F_OUT, DEG = 32, 12, 325, 64, 64, 16
BT = B * T
FA = 80
NP = 328
NC, NS = 2, 16
NW = NC * NS
BT_PER = BT // NW
ROWS = BT * N
RB = 2600
BLK = 4
CH = 192


def _tc_body(h_ref, w_ref, tab_ref, s2_ref):
    tab = jnp.dot(h_ref[...], w_ref[...], preferred_element_type=jnp.float32)
    tab_ref[...] = tab
    s2_ref[...] = tab[:, F_OUT:F_OUT + 1]


def _cnt_body(adj_ref, cnt_ref):
    adj = adj_ref[...]
    acc = jnp.zeros((N, DEG), jnp.int32)
    for j in range(DEG):
        acc = acc + jnp.where(adj == adj[:, j:j + 1], 1, 0)
    cnt_ref[...] = acc


def _sc_body(s2_hbm, pk_hbm, a_hbm, pk_v, s2a_v, a_v):
    cid = lax.axis_index("c")
    sid = lax.axis_index("s")
    wid = sid * NC + cid
    pltpu.sync_copy(pk_hbm, pk_v)
    pltpu.sync_copy(s2_hbm.at[wid], s2a_v)

    def zero_body(i, carry):
        z = jnp.zeros((16,), jnp.float32)
        for cb in range(NP // 16):
            a_v[i, pl.ds(cb * 16, 16)] = z
        a_v[i, pl.ds(NP - 16, 16)] = z
        return carry

    for off, rows in ((0, CH), (CH, NP - CH)):
        lax.fori_loop(0, CH, zero_body, 0)

        def bt_body(k, carry, off=off, rows=rows):
            bt = wid * BT_PER + k
            kvec = jnp.full((16,), k, jnp.int32)

            def node_body(i, carry2):
                g = jnp.minimum(off + i, N - 1)
                nbr = pk_v[g, pl.ds(0, DEG)]
                cnt = pk_v[g, pl.ds(DEG, DEG)]
                sv = plsc.load_gather(s2a_v, [kvec, nbr])
                m = jnp.max(sv)
                ex = jnp.exp(sv - m)
                asc = ex * cnt.astype(jnp.float32) / jnp.sum(ex)
                plsc.store_scatter(a_v, [jnp.full((16,), i, jnp.int32), nbr],
                                   asc)
                return carry2

            lax.fori_loop(0, rows, node_body, 0)
            if rows == CH:
                pltpu.sync_copy(a_v, a_hbm.at[bt, pl.ds(off, rows)])
            else:
                pltpu.sync_copy(a_v.at[pl.ds(0, rows)],
                                a_hbm.at[bt, pl.ds(off, rows)])
            return carry

        lax.fori_loop(0, BT_PER, bt_body, 0)


def _mm_body(a_ref, tab_ref, out_ref):
    hh = tab_ref[..., :F_OUT]
    res = lax.dot_general(
        a_ref[...], hh,
        dimension_numbers=(((2,), (1,)), ((0,), (0,))),
        preferred_element_type=jnp.float32)
    out_ref[...] = res[:, :N, :]


def kernel(h, adj, W, a):
    h2 = h.reshape(ROWS, F_IN)
    wT = W.T
    a2 = a[F_OUT:]
    waug = jnp.concatenate(
        [wT, (wT @ a2)[:, None], jnp.zeros((F_IN, FA - F_OUT - 1), jnp.float32)],
        axis=1)

    tab, s2c = pl.pallas_call(
        _tc_body,
        grid=(ROWS // RB,),
        in_specs=[
            pl.BlockSpec((RB, F_IN), lambda i: (i, 0)),
            pl.BlockSpec((F_IN, FA), lambda i: (0, 0)),
        ],
        out_specs=[
            pl.BlockSpec((RB, FA), lambda i: (i, 0)),
            pl.BlockSpec((RB, 1), lambda i: (i, 0)),
        ],
        out_shape=[
            jax.ShapeDtypeStruct((ROWS, FA), jnp.float32),
            jax.ShapeDtypeStruct((ROWS, 1), jnp.float32),
        ],
    )(h2, waug)

    cnt = pl.pallas_call(
        _cnt_body,
        out_shape=jax.ShapeDtypeStruct((N, DEG), jnp.int32),
    )(adj)
    pk = jnp.concatenate([adj, cnt], axis=1)

    sc_fn = pl.kernel(
        _sc_body,
        out_type=jax.ShapeDtypeStruct((BT, NP, NP), jnp.float32),
        mesh=plsc.VectorSubcoreMesh(core_axis_name="c", subcore_axis_name="s",
                                    num_cores=NC, num_subcores=NS),
        compiler_params=pltpu.CompilerParams(needs_layout_passes=False),
        scratch_types=[
            pltpu.VMEM((N, 2 * DEG), jnp.int32),
            pltpu.VMEM((BT_PER, N), jnp.float32),
            pltpu.VMEM((CH, NP), jnp.float32),
        ],
    )
    amat = sc_fn(s2c.reshape(NW, BT_PER, N), pk)

    tabp = jnp.pad(tab.reshape(BT, N, FA), ((0, 0), (0, NP - N), (0, 0)))

    outp = pl.pallas_call(
        _mm_body,
        grid=(BT // BLK,),
        in_specs=[
            pl.BlockSpec((BLK, NP, NP), lambda i: (i, 0, 0)),
            pl.BlockSpec((BLK, NP, FA), lambda i: (i, 0, 0)),
        ],
        out_specs=pl.BlockSpec((BLK, N, F_OUT), lambda i: (i, 0, 0)),
        out_shape=jax.ShapeDtypeStruct((BT, N, F_OUT), jnp.float32),
    )(amat, tabp)
    return outp.reshape(B, T, N, F_OUT)


# speedup vs baseline: 1.0054x; 1.0054x over previous
import jax
import jax.numpy as jnp
from jax import lax
from jax.experimental import pallas as pl
from jax.experimental.pallas import tpu as pltpu
from jax.experimental.pallas import tpu_sc as plsc

B, T, N, F_IN, F_OUT, DEG = 32, 12, 325, 64, 64, 16
BT = B * T
NP = 328
NC, NS = 2, 16
NW = NC * NS
BT_PER = BT // NW
TB = 8
BLK = 4
CH = 192
UNROLL = 4


def _tc_body(h_ref, w_ref, wa_ref, hh_ref, s2_ref):
    h = h_ref[...]
    hh_ref[...] = lax.dot_general(
        h, w_ref[...], dimension_numbers=(((2,), (0,)), ((), ())),
        preferred_element_type=jnp.float32)
    s2_ref[...] = lax.dot_general(
        h, wa_ref[...], dimension_numbers=(((2,), (0,)), ((), ())),
        preferred_element_type=jnp.float32)


def _cnt_body(adj_ref, cnt_ref):
    adj = adj_ref[...]
    acc = jnp.zeros((N, DEG), jnp.int32)
    for j in range(DEG):
        acc = acc + jnp.where(adj == adj[:, j:j + 1], 1, 0)
    cnt_ref[...] = acc


def _sc_body(s2_hbm, pk_hbm, a_hbm, pk_v, s2a_v, a_v):
    cid = lax.axis_index("c")
    sid = lax.axis_index("s")
    wid = sid * NC + cid
    bt0 = wid * BT_PER
    base = (bt0 // 8) * 8
    koff = bt0 - base
    pltpu.sync_copy(pk_hbm, pk_v)
    pltpu.sync_copy(s2_hbm.at[pl.ds(base, 16)], s2a_v)

    def zero_body(i, carry):
        z = jnp.zeros((16,), jnp.float32)
        for cb in range(NP // 16):
            a_v[i, pl.ds(cb * 16, 16)] = z
        a_v[i, pl.ds(NP - 16, 16)] = z
        return carry

    for off, rows in ((0, CH), (CH, NP - CH)):
        lax.fori_loop(0, CH, zero_body, 0)

        def bt_body(k, carry, off=off, rows=rows):
            bt = bt0 + k
            kvec = jnp.full((16,), koff + k, jnp.int32)

            def node_body(u, carry2):
                for v in range(UNROLL):
                    i = u * UNROLL + v
                    g = jnp.minimum(off + i, N - 1)
                    nbr = pk_v[g, pl.ds(0, DEG)]
                    cnt = pk_v[g, pl.ds(DEG, DEG)]
                    sv = plsc.load_gather(s2a_v, [kvec, nbr])
                    ex = jnp.exp(sv)
                    asc = ex * cnt.astype(jnp.float32) / jnp.sum(ex)
                    plsc.store_scatter(
                        a_v, [jnp.full((16,), i, jnp.int32), nbr], asc)
                return carry2

            lax.fori_loop(0, rows // UNROLL, node_body, 0)
            if rows == CH:
                pltpu.sync_copy(a_v, a_hbm.at[bt, pl.ds(off, rows)])
            else:
                pltpu.sync_copy(a_v.at[pl.ds(0, rows)],
                                a_hbm.at[bt, pl.ds(off, rows)])
            return carry

        lax.fori_loop(0, BT_PER, bt_body, 0)


def _mm_body(a_ref, hh_ref, out_ref):
    hh = jnp.pad(hh_ref[...], ((0, 0), (0, NP - N), (0, 0)))
    res = lax.dot_general(
        a_ref[...], hh,
        dimension_numbers=(((2,), (1,)), ((0,), (0,))),
        preferred_element_type=jnp.float32)
    out_ref[...] = res[:, :N, :]


def kernel(h, adj, W, a):
    h3 = h.reshape(BT, N, F_IN)
    wT = W.T
    wa2 = (wT @ a[F_OUT:])[:, None]

    hh, s2 = pl.pallas_call(
        _tc_body,
        grid=(BT // TB,),
        in_specs=[
            pl.BlockSpec((TB, N, F_IN), lambda i: (i, 0, 0)),
            pl.BlockSpec((F_IN, F_OUT), lambda i: (0, 0)),
            pl.BlockSpec((F_IN, 1), lambda i: (0, 0)),
        ],
        out_specs=[
            pl.BlockSpec((TB, N, F_OUT), lambda i: (i, 0, 0)),
            pl.BlockSpec((TB, N, 1), lambda i: (i, 0, 0)),
        ],
        out_shape=[
            jax.ShapeDtypeStruct((BT, N, F_OUT), jnp.float32),
            jax.ShapeDtypeStruct((BT, N, 1), jnp.float32),
        ],
    )(h3, wT, wa2)

    cnt = pl.pallas_call(
        _cnt_body,
        out_shape=jax.ShapeDtypeStruct((N, DEG), jnp.int32),
    )(adj)
    pk = jnp.concatenate([adj, cnt], axis=1)

    sc_fn = pl.kernel(
        _sc_body,
        out_type=jax.ShapeDtypeStruct((BT, NP, NP), jnp.float32),
        mesh=plsc.VectorSubcoreMesh(core_axis_name="c", subcore_axis_name="s",
                                    num_cores=NC, num_subcores=NS),
        compiler_params=pltpu.CompilerParams(needs_layout_passes=False),
        scratch_types=[
            pltpu.VMEM((N, 2 * DEG), jnp.int32),
            pltpu.VMEM((16, N), jnp.float32),
            pltpu.VMEM((CH, NP), jnp.float32),
        ],
    )
    amat = sc_fn(s2.reshape(BT, N), pk)

    outp = pl.pallas_call(
        _mm_body,
        grid=(BT // BLK,),
        in_specs=[
            pl.BlockSpec((BLK, NP, NP), lambda i: (i, 0, 0)),
            pl.BlockSpec((BLK, N, F_OUT), lambda i: (i, 0, 0)),
        ],
        out_specs=pl.BlockSpec((BLK, N, F_OUT), lambda i: (i, 0, 0)),
        out_shape=jax.ShapeDtypeStruct((BT, N, F_OUT), jnp.float32),
    )(amat, hh)
    return outp.reshape(B, T, N, F_OUT)


# speedup vs baseline: 1.0271x; 1.0215x over previous
import jax
import jax.numpy as jnp
from jax import lax
from jax.experimental import pallas as pl
from jax.experimental.pallas import tpu as pltpu
from jax.experimental.pallas import tpu_sc as plsc

B, T, N, F_IN, F_OUT, DEG = 32, 12, 325, 64, 64, 16
BT = B * T
FA = 80
NC, NS = 2, 16
NW = NC * NS
BT_PER = BT // NW
ROWS = BT * N
RB = 2600


def _tc_body(h_ref, w_ref, tab_ref):
    tab_ref[...] = jnp.dot(h_ref[...], w_ref[...],
                           preferred_element_type=jnp.float32)


def _sc_body(tab_hbm, adj_hbm, out_hbm, adj_v, tab_v, out_v):
    cid = lax.axis_index("c")
    sid = lax.axis_index("s")
    wid = sid * NC + cid
    pltpu.sync_copy(adj_hbm, adj_v)
    col_s2 = jnp.full((16,), F_OUT, jnp.int32)

    def bt_body(k, carry):
        bt = wid * BT_PER + k
        pltpu.sync_copy(tab_hbm.at[bt], tab_v)

        def node_body(i, carry2):
            nbr = adj_v[i, :]
            svals = plsc.load_gather(tab_v, [nbr, col_s2])
            m = jnp.max(svals)
            ex = jnp.exp(svals - m)
            alpha = ex / jnp.sum(ex)
            accs = [jnp.zeros((16,), jnp.float32) for _ in range(4)]
            for j in range(DEG):
                aj = alpha[j]
                ij = nbr[j]
                for cb in range(4):
                    accs[cb] = accs[cb] + aj * tab_v[ij, pl.ds(cb * 16, 16)]
            for cb in range(4):
                out_v[i, pl.ds(cb * 16, 16)] = accs[cb]
            return carry2

        lax.fori_loop(0, N, node_body, 0)
        pltpu.sync_copy(out_v, out_hbm.at[bt])
        return carry

    lax.fori_loop(0, BT_PER, bt_body, 0)


def kernel(h, adj, W, a):
    h2 = h.reshape(ROWS, F_IN)
    wT = W.T
    a2 = a[F_OUT:]
    waug = jnp.concatenate(
        [wT, (wT @ a2)[:, None], jnp.zeros((F_IN, FA - F_OUT - 1), jnp.float32)],
        axis=1)

    tab = pl.pallas_call(
        _tc_body,
        grid=(ROWS // RB,),
        in_specs=[
            pl.BlockSpec((RB, F_IN), lambda i: (i, 0)),
            pl.BlockSpec((F_IN, FA), lambda i: (0, 0)),
        ],
        out_specs=pl.BlockSpec((RB, FA), lambda i: (i, 0)),
        out_shape=jax.ShapeDtypeStruct((ROWS, FA), jnp.float32),
    )(h2, waug)

    sc_fn = pl.kernel(
        _sc_body,
        out_type=jax.ShapeDtypeStruct((BT, N, F_OUT), jnp.float32),
        mesh=plsc.VectorSubcoreMesh(core_axis_name="c", subcore_axis_name="s",
                                    num_cores=NC, num_subcores=NS),
        compiler_params=pltpu.CompilerParams(needs_layout_passes=False),
        scratch_types=[
            pltpu.VMEM((N, DEG), jnp.int32),
            pltpu.VMEM((N, FA), jnp.float32),
            pltpu.VMEM((N, F_OUT), jnp.float32),
        ],
    )
    outp = sc_fn(tab.reshape(BT, N, FA), adj)
    return outp.reshape(B, T, N, F_OUT)


# speedup vs baseline: 1.1556x; 1.1251x over previous
import jax
import jax.numpy as jnp
from jax import lax
from jax.experimental import pallas as pl
from jax.experimental.pallas import tpu as pltpu
from jax.experimental.pallas import tpu_sc as plsc

B, T, N, F_IN, F_OUT, DEG = 32, 12, 325, 64, 64, 16
BT = B * T
FA = 80
NC, NS = 2, 16
NW = NC * NS
BT_PER = BT // NW
ROWS = BT * N
RB = 2600


def _tc_body(h_ref, w_ref, tab_ref):
    tab_ref[...] = jnp.dot(h_ref[...], w_ref[...],
                           preferred_element_type=jnp.float32)


def _sc_body(tab_hbm, adj_hbm, out_hbm, adj_v, tab_v, out_v):
    cid = lax.axis_index("c")
    sid = lax.axis_index("s")
    wid = sid * NC + cid
    pltpu.sync_copy(adj_hbm, adj_v)
    col_s2 = jnp.full((16,), F_OUT, jnp.int32)

    def bt_body(k, carry):
        bt = wid * BT_PER + k
        pltpu.sync_copy(tab_hbm.at[bt], tab_v)

        def node_body(u, carry2):
            i0 = 2 * u
            i1 = jnp.minimum(i0 + 1, N - 1)
            nbr0 = adj_v[i0, :]
            nbr1 = adj_v[i1, :]
            sv0 = plsc.load_gather(tab_v, [nbr0, col_s2])
            sv1 = plsc.load_gather(tab_v, [nbr1, col_s2])
            ex0 = jnp.exp(sv0 - jnp.max(sv0))
            ex1 = jnp.exp(sv1 - jnp.max(sv1))
            ones = jnp.ones((16,), jnp.float32)
            inv0 = ones / jnp.sum(ex0)
            inv1 = ones / jnp.sum(ex1)
            a0 = [jnp.zeros((16,), jnp.float32) for _ in range(4)]
            a1 = [jnp.zeros((16,), jnp.float32) for _ in range(4)]
            for j in range(DEG):
                e0 = ex0[j]
                e1 = ex1[j]
                i0j = nbr0[j]
                i1j = nbr1[j]
                for cb in range(4):
                    a0[cb] = a0[cb] + e0 * tab_v[i0j, pl.ds(cb * 16, 16)]
                    a1[cb] = a1[cb] + e1 * tab_v[i1j, pl.ds(cb * 16, 16)]
            for cb in range(4):
                out_v[i0, pl.ds(cb * 16, 16)] = a0[cb] * inv0
                out_v[i1, pl.ds(cb * 16, 16)] = a1[cb] * inv1
            return carry2

        lax.fori_loop(0, (N + 1) // 2, node_body, 0)
        pltpu.sync_copy(out_v, out_hbm.at[bt])
        return carry

    lax.fori_loop(0, BT_PER, bt_body, 0)


def kernel(h, adj, W, a):
    h2 = h.reshape(ROWS, F_IN)
    wT = W.T
    a2 = a[F_OUT:]
    waug = jnp.concatenate(
        [wT, (wT @ a2)[:, None], jnp.zeros((F_IN, FA - F_OUT - 1), jnp.float32)],
        axis=1)

    tab = pl.pallas_call(
        _tc_body,
        grid=(ROWS // RB,),
        in_specs=[
            pl.BlockSpec((RB, F_IN), lambda i: (i, 0)),
            pl.BlockSpec((F_IN, FA), lambda i: (0, 0)),
        ],
        out_specs=pl.BlockSpec((RB, FA), lambda i: (i, 0)),
        out_shape=jax.ShapeDtypeStruct((ROWS, FA), jnp.float32),
    )(h2, waug)

    sc_fn = pl.kernel(
        _sc_body,
        out_type=jax.ShapeDtypeStruct((BT, N, F_OUT), jnp.float32),
        mesh=plsc.VectorSubcoreMesh(core_axis_name="c", subcore_axis_name="s",
                                    num_cores=NC, num_subcores=NS),
        compiler_params=pltpu.CompilerParams(needs_layout_passes=False),
        scratch_types=[
            pltpu.VMEM((N, DEG), jnp.int32),
            pltpu.VMEM((N, FA), jnp.float32),
            pltpu.VMEM((N, F_OUT), jnp.float32),
        ],
    )
    outp = sc_fn(tab.reshape(BT, N, FA), adj)
    return outp.reshape(B, T, N, F_OUT)


# speedup vs baseline: 1.1639x; 1.0072x over previous
import jax
import jax.numpy as jnp
from jax import lax
from jax.experimental import pallas as pl
from jax.experimental.pallas import tpu as pltpu
from jax.experimental.pallas import tpu_sc as plsc

B, T, N, F_IN, F_OUT, DEG = 32, 12, 325, 64, 64, 16
BT = B * T
FA = 80
NC, NS = 2, 16
NW = NC * NS
BT_PER = BT // NW
ROWS = BT * N
RB = 2600


def _tc_body(h_ref, w_ref, tab_ref):
    tab_ref[...] = jnp.dot(h_ref[...], w_ref[...],
                           preferred_element_type=jnp.float32)


def _sc_body(tab_hbm, adj_hbm, out_hbm, adj_v, tab_v, out_v):
    cid = lax.axis_index("c")
    sid = lax.axis_index("s")
    wid = sid * NC + cid
    pltpu.sync_copy(adj_hbm, adj_v)
    col_s2 = jnp.full((16,), F_OUT, jnp.int32)

    def bt_body(k, carry):
        bt = wid * BT_PER + k
        pltpu.sync_copy(tab_hbm.at[bt], tab_v)

        def node_body(u, carry2):
            idx = [jnp.minimum(4 * u + v, N - 1) for v in range(4)]
            nbrs = [adj_v[i, :] for i in idx]
            svs = [plsc.load_gather(tab_v, [nb, col_s2]) for nb in nbrs]
            exs = [jnp.exp(sv - jnp.max(sv)) for sv in svs]
            ones = jnp.ones((16,), jnp.float32)
            invs = [ones / jnp.sum(ex) for ex in exs]
            accs = [[jnp.zeros((16,), jnp.float32) for _ in range(4)]
                    for _ in range(4)]
            for j in range(DEG):
                for v in range(4):
                    ev = exs[v][j]
                    ivj = nbrs[v][j]
                    for cb in range(4):
                        accs[v][cb] = (accs[v][cb]
                                       + ev * tab_v[ivj, pl.ds(cb * 16, 16)])
            for v in range(4):
                for cb in range(4):
                    out_v[idx[v], pl.ds(cb * 16, 16)] = accs[v][cb] * invs[v]
            return carry2

        lax.fori_loop(0, (N + 3) // 4, node_body, 0)
        pltpu.sync_copy(out_v, out_hbm.at[bt])
        return carry

    lax.fori_loop(0, BT_PER, bt_body, 0)


def kernel(h, adj, W, a):
    h2 = h.reshape(ROWS, F_IN)
    wT = W.T
    a2 = a[F_OUT:]
    waug = jnp.concatenate(
        [wT, (wT @ a2)[:, None], jnp.zeros((F_IN, FA - F_OUT - 1), jnp.float32)],
        axis=1)

    tab = pl.pallas_call(
        _tc_body,
        grid=(ROWS // RB,),
        in_specs=[
            pl.BlockSpec((RB, F_IN), lambda i: (i, 0)),
            pl.BlockSpec((F_IN, FA), lambda i: (0, 0)),
        ],
        out_specs=pl.BlockSpec((RB, FA), lambda i: (i, 0)),
        out_shape=jax.ShapeDtypeStruct((ROWS, FA), jnp.float32),
    )(h2, waug)

    sc_fn = pl.kernel(
        _sc_body,
        out_type=jax.ShapeDtypeStruct((BT, N, F_OUT), jnp.float32),
        mesh=plsc.VectorSubcoreMesh(core_axis_name="c", subcore_axis_name="s",
                                    num_cores=NC, num_subcores=NS),
        compiler_params=pltpu.CompilerParams(needs_layout_passes=False),
        scratch_types=[
            pltpu.VMEM((N, DEG), jnp.int32),
            pltpu.VMEM((N, FA), jnp.float32),
            pltpu.VMEM((N, F_OUT), jnp.float32),
        ],
    )
    outp = sc_fn(tab.reshape(BT, N, FA), adj)
    return outp.reshape(B, T, N, F_OUT)
